# baseline (speedup 1.0000x reference)
import functools

import jax
import jax.numpy as jnp
from jax import lax
from jax.experimental import pallas as pl
from jax.experimental.pallas import tpu as pltpu
from jax.experimental.pallas import tpu_sc as plsc

N, E, D = 10000, 320000, 128
B, F = 4096, 64
GCN_UNITS, FUSION_UNITS, BASE_OUT = 128, 128, 64

NC, NS = 2, 16
NW = NC * NS
EPW = E // NW
K = 125
CHUNKS = EPW // K
ZC = 80
NZZ = N // ZC
ZZI = (NZZ + NS - 1) // NS
ZR = 200
NZC = N // ZR
ZITER = (NZC + NS - 1) // NS
BPW = B // NW

_mesh = plsc.VectorSubcoreMesh(core_axis_name="c", subcore_axis_name="s")


NPAD = 10240
DZR = 2048
NDZ = NPAD // DZR
KD = 80
CHD = EPW // KD


@functools.partial(
    pl.kernel,
    out_type=jax.ShapeDtypeStruct((NC, 1, NPAD), jnp.float32),
    mesh=_mesh,
    scratch_types=[
        pltpu.VMEM((CHD, KD), jnp.int32),
        pltpu.VMEM((KD,), jnp.float32),
        pltpu.VMEM((DZR,), jnp.float32),
        pltpu.VMEM_SHARED((NPAD,), jnp.float32),
    ],
)
def _deg_kernel(dst_hbm, deg_out, dst2d, onesb, zdb, dacc):
    c = lax.axis_index("c")
    s = lax.axis_index("s")
    w = c * NS + s
    z16 = jnp.zeros((16,), jnp.float32)
    o16 = jnp.ones((16,), jnp.float32)

    def ones_body(k, _):
        onesb[pl.ds(k * 16, 16)] = o16
        return 0
    lax.fori_loop(0, KD // 16, ones_body, 0)

    def zero_body(j, _):
        zdb[pl.ds(j * 16, 16)] = z16
        return 0
    lax.fori_loop(0, DZR // 16, zero_body, 0)

    @pl.when(s < NDZ)
    def _():
        pltpu.sync_copy(zdb, dacc.at[pl.ds(s * DZR, DZR)])

    pltpu.sync_copy(dst_hbm.at[w], dst2d)

    plsc.subcore_barrier()

    def chunk_body(i, _):
        pltpu.sync_copy(onesb, dacc.at[dst2d.at[i]], add=True)
        return 0
    lax.fori_loop(0, CHD, chunk_body, 0)

    plsc.subcore_barrier()

    @pl.when(s < NDZ)
    def _():
        pltpu.sync_copy(dacc.at[pl.ds(s * DZR, DZR)],
                        deg_out.at[c, 0, pl.ds(s * DZR, DZR)])


KP = 128
NACC = N + 8


@functools.partial(
    pl.kernel,
    out_type=jax.ShapeDtypeStruct((NC, N, D), jnp.float32),
    mesh=_mesh,
    scratch_types=[
        pltpu.VMEM((CHUNKS, KP), jnp.int32),
        pltpu.VMEM((KP,), jnp.int32),
        pltpu.VMEM((KP,), jnp.int32),
        pltpu.VMEM((KP, D), jnp.float32),
        pltpu.VMEM((KP, D), jnp.float32),
        pltpu.VMEM_SHARED((NACC, D), jnp.float32),
        pltpu.SemaphoreType.DMA,
        pltpu.SemaphoreType.DMA,
        pltpu.SemaphoreType.DMA,
        pltpu.SemaphoreType.DMA,
    ],
)
def _prop_kernel(x_hbm, src_hbm, dstf_hbm, out_hbm, src2d, dstb0, dstb1,
                 rows0, rows1, acc, sg0, sg1, sd0, sd1):
    c = lax.axis_index("c")
    s = lax.axis_index("s")
    w = c * NS + s
    z16 = jnp.zeros((16,), jnp.float32)

    def zzero(i, _):
        for k in range(D // 16):
            rows0[i, pl.ds(k * 16, 16)] = z16
        return 0
    lax.fori_loop(0, KP, zzero, 0)

    for t in range(ZZI):
        u = t * NS + s

        @pl.when(u < NZZ)
        def _():
            pltpu.sync_copy(rows0.at[pl.ds(0, ZC)], acc.at[pl.ds(u * ZC, ZC)])

    pltpu.sync_copy(src_hbm.at[w], src2d)

    def start(j, rb, db, sg, sd):
        pltpu.async_copy(dstf_hbm.at[pl.ds((w * CHUNKS + j) * KP, KP)], db, sd)
        pltpu.async_copy(x_hbm.at[src2d.at[j]], rb, sg)

    def wait(j, rb, db, sg, sd):
        pltpu.make_async_copy(
            dstf_hbm.at[pl.ds((w * CHUNKS + j) * KP, KP)], db, sd).wait()
        pltpu.make_async_copy(x_hbm.at[src2d.at[j]], rb, sg).wait()

    start(0, rows0, dstb0, sg0, sd0)
    start(1, rows1, dstb1, sg1, sd1)

    plsc.subcore_barrier()

    def chunk_body(t, _):
        j = 2 * t
        wait(j, rows0, dstb0, sg0, sd0)
        pltpu.sync_copy(rows0, acc.at[dstb0], add=True)

        @pl.when(j + 2 < CHUNKS)
        def _():
            start(j + 2, rows0, dstb0, sg0, sd0)

        wait(j + 1, rows1, dstb1, sg1, sd1)
        pltpu.sync_copy(rows1, acc.at[dstb1], add=True)

        @pl.when(j + 3 < CHUNKS)
        def _():
            start(j + 3, rows1, dstb1, sg1, sd1)
        return 0
    lax.fori_loop(0, CHUNKS // 2, chunk_body, 0)

    plsc.subcore_barrier()

    for t in range(ZITER):
        u = t * NS + s

        @pl.when(u < NZC)
        def _():
            pltpu.sync_copy(acc.at[pl.ds(u * ZR, ZR)],
                            out_hbm.at[c, pl.ds(u * ZR, ZR)])


@functools.partial(
    pl.kernel,
    out_type=[
        jax.ShapeDtypeStruct((B, D), jnp.float32),
        jax.ShapeDtypeStruct((B, D), jnp.float32),
        jax.ShapeDtypeStruct((B,), jnp.float32),
    ],
    mesh=_mesh,
    scratch_types=[
        pltpu.VMEM((BPW,), jnp.float32),
        pltpu.VMEM((BPW,), jnp.int32),
        pltpu.VMEM((BPW, D), jnp.float32),
        pltpu.VMEM((BPW,), jnp.float32),
        pltpu.SemaphoreType.DMA,
    ],
)
def _gather_kernel(mid_hbm, q_hbm, dinv_hbm, g0_out, g1_out, dv_out,
                   mv, idxv, rowbuf, dvv, sem):
    c = lax.axis_index("c")
    s = lax.axis_index("s")
    w = c * NS + s
    base = w * BPW
    pltpu.sync_copy(mid_hbm.at[pl.ds(base, BPW)], mv)

    def idx_body(g, _):
        v = mv[pl.ds(g * 16, 16)]
        idxv[pl.ds(g * 16, 16)] = (v * jnp.float32(N - 1)).astype(jnp.int32)
        return 0
    lax.fori_loop(0, BPW // 16, idx_body, 0)

    pltpu.async_copy(q_hbm.at[0].at[idxv], rowbuf, sem).wait()
    pltpu.sync_copy(rowbuf, g0_out.at[pl.ds(base, BPW)])
    pltpu.async_copy(q_hbm.at[1].at[idxv], rowbuf, sem).wait()
    pltpu.sync_copy(rowbuf, g1_out.at[pl.ds(base, BPW)])

    pltpu.async_copy(dinv_hbm.at[idxv], dvv, sem).wait()
    pltpu.sync_copy(dvv, dv_out.at[pl.ds(base, BPW)])


def _t1_body(p_ref, nf_ref, dinv_ref, xt0_ref):
    deg = jnp.maximum(jnp.sum(p_ref[:, :N], axis=0), 1.0)
    dinv = lax.rsqrt(deg)
    dinv_ref[...] = dinv
    xt0_ref[...] = nf_ref[...] * dinv[:, None]


def _t2_body(p_ref, dinv_ref, w_ref, b_ref, xt1_ref):
    dinv = dinv_ref[...]
    agg = (p_ref[0] + p_ref[1]) * dinv[:, None]
    x1 = jax.nn.relu(
        jnp.dot(agg, w_ref[...], preferred_element_type=jnp.float32)
        + b_ref[...][None, :])
    xt1_ref[...] = x1 * dinv[:, None]


def _t3_body(f_ref, g0_ref, g1_ref, dvg_ref, wb_ref, bb_ref, w1_ref, b1_ref,
             wf_ref, bf_ref, wo_ref, bo_ref, out_ref):
    base = jax.nn.relu(
        jnp.dot(f_ref[...], wb_ref[...], preferred_element_type=jnp.float32)
        + bb_ref[...][None, :])
    dvg = dvg_ref[...]
    aggg = (g0_ref[...] + g1_ref[...]) * dvg[:, None]
    graph = jax.nn.relu(
        jnp.dot(aggg, w1_ref[...], preferred_element_type=jnp.float32)
        + b1_ref[...][None, :])
    fusion = jax.nn.relu(
        jnp.dot(base, wf_ref[0:BASE_OUT, :], preferred_element_type=jnp.float32)
        + jnp.dot(graph, wf_ref[BASE_OUT:, :], preferred_element_type=jnp.float32)
        + bf_ref[...][None, :])
    out_ref[...] = jax.nn.sigmoid(
        jnp.dot(fusion, wo_ref[...], preferred_element_type=jnp.float32)
        + bo_ref[...][None, :])


def kernel(movieId, features, node_features, edge_index, Wb, bb, W_gcn0,
           b_gcn0, W_gcn1, b_gcn1, Wf, bf, Wo, bo):
    src3 = edge_index[0].astype(jnp.int32).reshape(NW, CHUNKS, K)
    dst3 = edge_index[1].astype(jnp.int32).reshape(NW, CHUNKS, K)
    src = jnp.pad(src3, ((0, 0), (0, 0), (0, KP - K)))
    dstf = jnp.pad(dst3, ((0, 0), (0, 0), (0, KP - K)),
                   constant_values=N).reshape(-1)

    dst_d = edge_index[1].astype(jnp.int32).reshape(NW, CHD, KD)
    deg_partials = _deg_kernel(dst_d).reshape(NC, NPAD)

    dinv, xt0 = pl.pallas_call(
        _t1_body,
        out_shape=[
            jax.ShapeDtypeStruct((N,), jnp.float32),
            jax.ShapeDtypeStruct((N, D), jnp.float32),
        ],
    )(deg_partials, node_features)

    p = _prop_kernel(xt0, src, dstf)

    xt1 = pl.pallas_call(
        _t2_body,
        out_shape=jax.ShapeDtypeStruct((N, GCN_UNITS), jnp.float32),
    )(p, dinv, W_gcn0, b_gcn0)

    q = _prop_kernel(xt1, src, dstf)

    g0, g1, dvg = _gather_kernel(movieId, q, dinv)

    out = pl.pallas_call(
        _t3_body,
        out_shape=jax.ShapeDtypeStruct((B, 1), jnp.float32),
    )(features, g0, g1, dvg, Wb, bb, W_gcn1, b_gcn1, Wf, bf, Wo, bo)
    return out


# speedup vs baseline: 3.0343x; 3.0343x over previous
import functools

import jax
import jax.numpy as jnp
from jax import lax
from jax.experimental import pallas as pl
from jax.experimental.pallas import tpu as pltpu
from jax.experimental.pallas import tpu_sc as plsc

N, E, D = 10000, 320000, 128
B, F = 4096, 64
GCN_UNITS, FUSION_UNITS, BASE_OUT = 128, 128, 64

NC, NS = 2, 16
NW = NC * NS
EPW = E // NW
K = 125
CHUNKS = EPW // K
ZC = 80
NZZ = N // ZC
ZZI = (NZZ + NS - 1) // NS
ZR = 200
NZC = N // ZR
ZITER = (NZC + NS - 1) // NS
BPW = B // NW

_mesh = plsc.VectorSubcoreMesh(core_axis_name="c", subcore_axis_name="s")


NPAD = 10240
DZR = 2048
NDZ = NPAD // DZR
KD = 80
CHD = EPW // KD


@functools.partial(
    pl.kernel,
    out_type=jax.ShapeDtypeStruct((NC, 1, NPAD), jnp.float32),
    mesh=_mesh,
    scratch_types=[
        pltpu.VMEM((CHD, KD), jnp.int32),
        pltpu.VMEM((KD,), jnp.float32),
        pltpu.VMEM((DZR,), jnp.float32),
        pltpu.VMEM_SHARED((NPAD,), jnp.float32),
    ],
)
def _deg_kernel(dst_hbm, deg_out, dst2d, onesb, zdb, dacc):
    c = lax.axis_index("c")
    s = lax.axis_index("s")
    w = c * NS + s
    z16 = jnp.zeros((16,), jnp.float32)
    o16 = jnp.ones((16,), jnp.float32)

    def ones_body(k, _):
        onesb[pl.ds(k * 16, 16)] = o16
        return 0
    lax.fori_loop(0, KD // 16, ones_body, 0)

    def zero_body(j, _):
        zdb[pl.ds(j * 16, 16)] = z16
        return 0
    lax.fori_loop(0, DZR // 16, zero_body, 0)

    @pl.when(s < NDZ)
    def _():
        pltpu.sync_copy(zdb, dacc.at[pl.ds(s * DZR, DZR)])

    pltpu.sync_copy(dst_hbm.at[w], dst2d)

    plsc.subcore_barrier()

    def chunk_body(i, _):
        pltpu.sync_copy(onesb, dacc.at[dst2d.at[i]], add=True)
        return 0
    lax.fori_loop(0, CHD, chunk_body, 0)

    plsc.subcore_barrier()

    @pl.when(s < NDZ)
    def _():
        pltpu.sync_copy(dacc.at[pl.ds(s * DZR, DZR)],
                        deg_out.at[c, 0, pl.ds(s * DZR, DZR)])


HCH = CHUNKS // 2


@functools.partial(
    pl.kernel,
    out_type=jax.ShapeDtypeStruct((NC, N, D), jnp.float32),
    mesh=_mesh,
    scratch_types=[
        pltpu.VMEM((HCH, K), jnp.int32),
        pltpu.VMEM((CHUNKS, K), jnp.int32),
        pltpu.VMEM((K, D), jnp.float32),
        pltpu.VMEM((K, D), jnp.float32),
        pltpu.VMEM_SHARED((N, D), jnp.float32),
        pltpu.SemaphoreType.DMA,
        pltpu.SemaphoreType.DMA,
    ],
)
def _prop_kernel(x_hbm, src_hbm, dst_hbm, out_hbm, src2d, dst2d,
                 rows0, rows1, acc, sg0, sg1):
    c = lax.axis_index("c")
    s = lax.axis_index("s")
    w = c * NS + s
    z16 = jnp.zeros((16,), jnp.float32)

    def zzero(i, _):
        for k in range(D // 16):
            rows0[i, pl.ds(k * 16, 16)] = z16
        return 0
    lax.fori_loop(0, K, zzero, 0)

    for t in range(ZZI):
        u = t * NS + s

        @pl.when(u < NZZ)
        def _():
            pltpu.sync_copy(rows0.at[pl.ds(0, ZC)], acc.at[pl.ds(u * ZC, ZC)])

    pltpu.sync_copy(dst_hbm.at[w], dst2d)
    pltpu.sync_copy(src_hbm.at[w, pl.ds(0, HCH)], src2d)

    plsc.subcore_barrier()

    def gstart(jj, rb, sg):
        pltpu.async_copy(x_hbm.at[src2d.at[jj]], rb, sg)

    def gwait(jj, rb, sg):
        pltpu.make_async_copy(x_hbm.at[src2d.at[jj]], rb, sg).wait()

    for p in range(2):
        jbase = p * HCH

        @pl.when(jnp.bool_(p > 0))
        def _():
            pltpu.sync_copy(src_hbm.at[w, pl.ds(p * HCH, HCH)], src2d)

        gstart(0, rows0, sg0)
        gstart(1, rows1, sg1)

        def chunk_body(t, _):
            jj = 2 * t
            gwait(jj, rows0, sg0)

            @pl.when(jj + 2 < HCH)
            def _():
                gstart(jj + 2, rows0, sg0)
            pltpu.sync_copy(rows0, acc.at[dst2d.at[jbase + jj]], add=True)

            gwait(jj + 1, rows1, sg1)

            @pl.when(jj + 3 < HCH)
            def _():
                gstart(jj + 3, rows1, sg1)
            pltpu.sync_copy(rows1, acc.at[dst2d.at[jbase + jj + 1]], add=True)
            return 0
        lax.fori_loop(0, HCH // 2, chunk_body, 0)

    plsc.subcore_barrier()

    for t in range(ZITER):
        u = t * NS + s

        @pl.when(u < NZC)
        def _():
            pltpu.sync_copy(acc.at[pl.ds(u * ZR, ZR)],
                            out_hbm.at[c, pl.ds(u * ZR, ZR)])


@functools.partial(
    pl.kernel,
    out_type=[
        jax.ShapeDtypeStruct((B, D), jnp.float32),
        jax.ShapeDtypeStruct((B, D), jnp.float32),
        jax.ShapeDtypeStruct((B,), jnp.float32),
    ],
    mesh=_mesh,
    scratch_types=[
        pltpu.VMEM((BPW,), jnp.float32),
        pltpu.VMEM((BPW,), jnp.int32),
        pltpu.VMEM((BPW, D), jnp.float32),
        pltpu.VMEM((BPW,), jnp.float32),
        pltpu.SemaphoreType.DMA,
    ],
)
def _gather_kernel(mid_hbm, q_hbm, dinv_hbm, g0_out, g1_out, dv_out,
                   mv, idxv, rowbuf, dvv, sem):
    c = lax.axis_index("c")
    s = lax.axis_index("s")
    w = c * NS + s
    base = w * BPW
    pltpu.sync_copy(mid_hbm.at[pl.ds(base, BPW)], mv)

    def idx_body(g, _):
        v = mv[pl.ds(g * 16, 16)]
        idxv[pl.ds(g * 16, 16)] = (v * jnp.float32(N - 1)).astype(jnp.int32)
        return 0
    lax.fori_loop(0, BPW // 16, idx_body, 0)

    pltpu.async_copy(q_hbm.at[0].at[idxv], rowbuf, sem).wait()
    pltpu.sync_copy(rowbuf, g0_out.at[pl.ds(base, BPW)])
    pltpu.async_copy(q_hbm.at[1].at[idxv], rowbuf, sem).wait()
    pltpu.sync_copy(rowbuf, g1_out.at[pl.ds(base, BPW)])

    pltpu.async_copy(dinv_hbm.at[idxv], dvv, sem).wait()
    pltpu.sync_copy(dvv, dv_out.at[pl.ds(base, BPW)])


def _t1_body(p_ref, nf_ref, dinv_ref, xt0_ref):
    deg = jnp.maximum(jnp.sum(p_ref[:, :N], axis=0), 1.0)
    dinv = lax.rsqrt(deg)
    dinv_ref[...] = dinv
    xt0_ref[...] = nf_ref[...] * dinv[:, None]


def _t2_body(p_ref, dinv_ref, w_ref, b_ref, xt1_ref):
    dinv = dinv_ref[...]
    agg = (p_ref[0] + p_ref[1]) * dinv[:, None]
    x1 = jax.nn.relu(
        jnp.dot(agg, w_ref[...], preferred_element_type=jnp.float32)
        + b_ref[...][None, :])
    xt1_ref[...] = x1 * dinv[:, None]


def _t3_body(f_ref, g0_ref, g1_ref, dvg_ref, wb_ref, bb_ref, w1_ref, b1_ref,
             wf_ref, bf_ref, wo_ref, bo_ref, out_ref):
    base = jax.nn.relu(
        jnp.dot(f_ref[...], wb_ref[...], preferred_element_type=jnp.float32)
        + bb_ref[...][None, :])
    dvg = dvg_ref[...]
    aggg = (g0_ref[...] + g1_ref[...]) * dvg[:, None]
    graph = jax.nn.relu(
        jnp.dot(aggg, w1_ref[...], preferred_element_type=jnp.float32)
        + b1_ref[...][None, :])
    fusion = jax.nn.relu(
        jnp.dot(base, wf_ref[0:BASE_OUT, :], preferred_element_type=jnp.float32)
        + jnp.dot(graph, wf_ref[BASE_OUT:, :], preferred_element_type=jnp.float32)
        + bf_ref[...][None, :])
    out_ref[...] = jax.nn.sigmoid(
        jnp.dot(fusion, wo_ref[...], preferred_element_type=jnp.float32)
        + bo_ref[...][None, :])


def kernel(movieId, features, node_features, edge_index, Wb, bb, W_gcn0,
           b_gcn0, W_gcn1, b_gcn1, Wf, bf, Wo, bo):
    src = edge_index[0].astype(jnp.int32).reshape(NW, CHUNKS, K)
    dst = edge_index[1].astype(jnp.int32).reshape(NW, CHUNKS, K)

    dst_d = edge_index[1].astype(jnp.int32).reshape(NW, CHD, KD)
    deg_partials = _deg_kernel(dst_d).reshape(NC, NPAD)

    dinv, xt0 = pl.pallas_call(
        _t1_body,
        out_shape=[
            jax.ShapeDtypeStruct((N,), jnp.float32),
            jax.ShapeDtypeStruct((N, D), jnp.float32),
        ],
    )(deg_partials, node_features)

    p = _prop_kernel(xt0, src, dst)

    xt1 = pl.pallas_call(
        _t2_body,
        out_shape=jax.ShapeDtypeStruct((N, GCN_UNITS), jnp.float32),
    )(p, dinv, W_gcn0, b_gcn0)

    q = _prop_kernel(xt1, src, dst)

    g0, g1, dvg = _gather_kernel(movieId, q, dinv)

    out = pl.pallas_call(
        _t3_body,
        out_shape=jax.ShapeDtypeStruct((B, 1), jnp.float32),
    )(features, g0, g1, dvg, Wb, bb, W_gcn1, b_gcn1, Wf, bf, Wo, bo)
    return out


# speedup vs baseline: 3.1973x; 1.0537x over previous
import functools

import jax
import jax.numpy as jnp
from jax import lax
from jax.experimental import pallas as pl
from jax.experimental.pallas import tpu as pltpu
from jax.experimental.pallas import tpu_sc as plsc

N, E, D = 10000, 320000, 128
B, F = 4096, 64
GCN_UNITS, FUSION_UNITS, BASE_OUT = 128, 128, 64

NC, NS = 2, 16
NW = NC * NS
EPW = E // NW
K = 125
CHUNKS = EPW // K
ZC = 80
NZZ = N // ZC
ZZI = (NZZ + NS - 1) // NS
ZR = 200
NZC = N // ZR
ZITER = (NZC + NS - 1) // NS
BPW = B // NW

_mesh = plsc.VectorSubcoreMesh(core_axis_name="c", subcore_axis_name="s")


NPAD = 10240
DZR = 2048
NDZ = NPAD // DZR
KD = 80
CHD = EPW // KD


@functools.partial(
    pl.kernel,
    out_type=jax.ShapeDtypeStruct((NC, 1, NPAD), jnp.float32),
    mesh=_mesh,
    scratch_types=[
        pltpu.VMEM((CHD, KD), jnp.int32),
        pltpu.VMEM((KD,), jnp.float32),
        pltpu.VMEM((DZR,), jnp.float32),
        pltpu.VMEM_SHARED((NPAD,), jnp.float32),
        pltpu.SemaphoreType.DMA,
    ],
)
def _deg_kernel(dst_hbm, deg_out, dst2d, onesb, zdb, dacc, sda):
    c = lax.axis_index("c")
    s = lax.axis_index("s")
    w = c * NS + s
    z16 = jnp.zeros((16,), jnp.float32)
    o16 = jnp.ones((16,), jnp.float32)

    def ones_body(k, _):
        onesb[pl.ds(k * 16, 16)] = o16
        return 0
    lax.fori_loop(0, KD // 16, ones_body, 0)

    def zero_body(j, _):
        zdb[pl.ds(j * 16, 16)] = z16
        return 0
    lax.fori_loop(0, DZR // 16, zero_body, 0)

    @pl.when(s < NDZ)
    def _():
        pltpu.sync_copy(zdb, dacc.at[pl.ds(s * DZR, DZR)])

    pltpu.sync_copy(dst_hbm.at[w], dst2d)

    plsc.subcore_barrier()

    def chunk_body(i, _):
        pltpu.async_copy(onesb, dacc.at[dst2d.at[i]], sda, add=True)
        return 0
    lax.fori_loop(0, CHD, chunk_body, 0)

    def drain_body(i, _):
        pltpu.make_async_copy(onesb, dacc.at[dst2d.at[i]], sda).wait()
        return 0
    lax.fori_loop(0, CHD, drain_body, 0)

    plsc.subcore_barrier()

    @pl.when(s < NDZ)
    def _():
        pltpu.sync_copy(dacc.at[pl.ds(s * DZR, DZR)],
                        deg_out.at[c, 0, pl.ds(s * DZR, DZR)])


HCH = CHUNKS // 2


@functools.partial(
    pl.kernel,
    out_type=jax.ShapeDtypeStruct((NC, N, D), jnp.float32),
    mesh=_mesh,
    scratch_types=[
        pltpu.VMEM((HCH, K), jnp.int32),
        pltpu.VMEM((CHUNKS, K), jnp.int32),
        pltpu.VMEM((K, D), jnp.float32),
        pltpu.VMEM((K, D), jnp.float32),
        pltpu.VMEM_SHARED((N, D), jnp.float32),
        pltpu.SemaphoreType.DMA,
        pltpu.SemaphoreType.DMA,
    ],
)
def _prop_kernel(x_hbm, src_hbm, dst_hbm, out_hbm, src2d, dst2d,
                 rows0, rows1, acc, sg0, sg1):
    c = lax.axis_index("c")
    s = lax.axis_index("s")
    w = c * NS + s
    z16 = jnp.zeros((16,), jnp.float32)

    def zzero(i, _):
        for k in range(D // 16):
            rows0[i, pl.ds(k * 16, 16)] = z16
        return 0
    lax.fori_loop(0, K, zzero, 0)

    for t in range(ZZI):
        u = t * NS + s

        @pl.when(u < NZZ)
        def _():
            pltpu.sync_copy(rows0.at[pl.ds(0, ZC)], acc.at[pl.ds(u * ZC, ZC)])

    pltpu.sync_copy(dst_hbm.at[w], dst2d)
    pltpu.sync_copy(src_hbm.at[w, pl.ds(0, HCH)], src2d)

    plsc.subcore_barrier()

    def gstart(jj, rb, sg):
        pltpu.async_copy(x_hbm.at[src2d.at[jj]], rb, sg)

    def gwait(jj, rb, sg):
        pltpu.make_async_copy(x_hbm.at[src2d.at[jj]], rb, sg).wait()

    for p in range(2):
        jbase = p * HCH

        @pl.when(jnp.bool_(p > 0))
        def _():
            pltpu.sync_copy(src_hbm.at[w, pl.ds(p * HCH, HCH)], src2d)

        gstart(0, rows0, sg0)
        gstart(1, rows1, sg1)

        def chunk_body(t, _):
            jj = 2 * t
            gwait(jj, rows0, sg0)

            @pl.when(jj + 2 < HCH)
            def _():
                gstart(jj + 2, rows0, sg0)
            pltpu.sync_copy(rows0, acc.at[dst2d.at[jbase + jj]], add=True)

            gwait(jj + 1, rows1, sg1)

            @pl.when(jj + 3 < HCH)
            def _():
                gstart(jj + 3, rows1, sg1)
            pltpu.sync_copy(rows1, acc.at[dst2d.at[jbase + jj + 1]], add=True)
            return 0
        lax.fori_loop(0, HCH // 2, chunk_body, 0)

    plsc.subcore_barrier()

    for t in range(ZITER):
        u = t * NS + s

        @pl.when(u < NZC)
        def _():
            pltpu.sync_copy(acc.at[pl.ds(u * ZR, ZR)],
                            out_hbm.at[c, pl.ds(u * ZR, ZR)])


BPC = B // NS


@functools.partial(
    pl.kernel,
    out_type=[
        jax.ShapeDtypeStruct((NC, B, D), jnp.float32),
        jax.ShapeDtypeStruct((B,), jnp.float32),
    ],
    mesh=_mesh,
    scratch_types=[
        pltpu.VMEM((HCH, K), jnp.int32),
        pltpu.VMEM((CHUNKS, K), jnp.int32),
        pltpu.VMEM((128, D), jnp.float32),
        pltpu.VMEM((128, D), jnp.float32),
        pltpu.VMEM((BPC,), jnp.float32),
        pltpu.VMEM((2, 128), jnp.int32),
        pltpu.VMEM((128,), jnp.float32),
        pltpu.VMEM_SHARED((N, D), jnp.float32),
        pltpu.SemaphoreType.DMA,
        pltpu.SemaphoreType.DMA,
    ],
)
def _prop2_kernel(x_hbm, src_hbm, dst_hbm, mid_hbm, dinv_hbm, g_out, dv_out,
                  src2d, dst2d, rows0, rows1, mv, idx2, dvb, acc, sg0, sg1):
    c = lax.axis_index("c")
    s = lax.axis_index("s")
    w = c * NS + s
    z16 = jnp.zeros((16,), jnp.float32)

    def zzero(i, _):
        for k in range(D // 16):
            rows0[i, pl.ds(k * 16, 16)] = z16
        return 0
    lax.fori_loop(0, 128, zzero, 0)

    for t in range(ZZI):
        u = t * NS + s

        @pl.when(u < NZZ)
        def _():
            pltpu.sync_copy(rows0.at[pl.ds(0, ZC)], acc.at[pl.ds(u * ZC, ZC)])

    pltpu.sync_copy(dst_hbm.at[w], dst2d)
    pltpu.sync_copy(src_hbm.at[w, pl.ds(0, HCH)], src2d)
    pltpu.sync_copy(mid_hbm.at[pl.ds(s * BPC, BPC)], mv)

    for g in range(BPC // 16):
        v = mv[pl.ds(g * 16, 16)]
        idx2[g // 8, pl.ds((g % 8) * 16, 16)] = (
            v * jnp.float32(N - 1)).astype(jnp.int32)

    plsc.subcore_barrier()

    def gstart(jj, rb, sg):
        pltpu.async_copy(x_hbm.at[src2d.at[jj]], rb.at[pl.ds(0, K)], sg)

    def gwait(jj, rb, sg):
        pltpu.make_async_copy(x_hbm.at[src2d.at[jj]], rb.at[pl.ds(0, K)],
                              sg).wait()

    for p in range(2):
        jbase = p * HCH

        @pl.when(jnp.bool_(p > 0))
        def _():
            pltpu.sync_copy(src_hbm.at[w, pl.ds(p * HCH, HCH)], src2d)

        gstart(0, rows0, sg0)
        gstart(1, rows1, sg1)

        def chunk_body(t, _):
            jj = 2 * t
            gwait(jj, rows0, sg0)

            @pl.when(jj + 2 < HCH)
            def _():
                gstart(jj + 2, rows0, sg0)
            pltpu.sync_copy(rows0.at[pl.ds(0, K)], acc.at[dst2d.at[jbase + jj]], add=True)

            gwait(jj + 1, rows1, sg1)

            @pl.when(jj + 3 < HCH)
            def _():
                gstart(jj + 3, rows1, sg1)
            pltpu.sync_copy(rows1.at[pl.ds(0, K)], acc.at[dst2d.at[jbase + jj + 1]], add=True)
            return 0
        lax.fori_loop(0, HCH // 2, chunk_body, 0)

    plsc.subcore_barrier()

    for r in range(2):
        pltpu.async_copy(acc.at[idx2.at[r]], rows0, sg0)
        pltpu.make_async_copy(acc.at[idx2.at[r]], rows0, sg0).wait()
        pltpu.sync_copy(rows0,
                        g_out.at[c, pl.ds(s * BPC + r * 128, 128)])

    @pl.when(c == 0)
    def _():
        for r in range(2):
            pltpu.async_copy(dinv_hbm.at[idx2.at[r]], dvb, sg1)
            pltpu.make_async_copy(dinv_hbm.at[idx2.at[r]], dvb, sg1).wait()
            pltpu.sync_copy(dvb, dv_out.at[pl.ds(s * BPC + r * 128, 128)])


@functools.partial(
    pl.kernel,
    out_type=[
        jax.ShapeDtypeStruct((B, D), jnp.float32),
        jax.ShapeDtypeStruct((B, D), jnp.float32),
        jax.ShapeDtypeStruct((B,), jnp.float32),
    ],
    mesh=_mesh,
    scratch_types=[
        pltpu.VMEM((BPW,), jnp.float32),
        pltpu.VMEM((BPW,), jnp.int32),
        pltpu.VMEM((BPW, D), jnp.float32),
        pltpu.VMEM((BPW,), jnp.float32),
        pltpu.SemaphoreType.DMA,
    ],
)
def _gather_kernel(mid_hbm, q_hbm, dinv_hbm, g0_out, g1_out, dv_out,
                   mv, idxv, rowbuf, dvv, sem):
    c = lax.axis_index("c")
    s = lax.axis_index("s")
    w = c * NS + s
    base = w * BPW
    pltpu.sync_copy(mid_hbm.at[pl.ds(base, BPW)], mv)

    def idx_body(g, _):
        v = mv[pl.ds(g * 16, 16)]
        idxv[pl.ds(g * 16, 16)] = (v * jnp.float32(N - 1)).astype(jnp.int32)
        return 0
    lax.fori_loop(0, BPW // 16, idx_body, 0)

    pltpu.async_copy(q_hbm.at[0].at[idxv], rowbuf, sem).wait()
    pltpu.sync_copy(rowbuf, g0_out.at[pl.ds(base, BPW)])
    pltpu.async_copy(q_hbm.at[1].at[idxv], rowbuf, sem).wait()
    pltpu.sync_copy(rowbuf, g1_out.at[pl.ds(base, BPW)])

    pltpu.async_copy(dinv_hbm.at[idxv], dvv, sem).wait()
    pltpu.sync_copy(dvv, dv_out.at[pl.ds(base, BPW)])


def _t1_body(p_ref, nf_ref, dinv_ref, xt0_ref):
    deg = jnp.maximum(jnp.sum(p_ref[:, :N], axis=0), 1.0)
    dinv = lax.rsqrt(deg)
    dinv_ref[...] = dinv
    xt0_ref[...] = nf_ref[...] * dinv[:, None]


def _t2_body(p_ref, dinv_ref, w_ref, b_ref, xt1_ref):
    dinv = dinv_ref[...]
    agg = (p_ref[0] + p_ref[1]) * dinv[:, None]
    x1 = jax.nn.relu(
        jnp.dot(agg, w_ref[...], preferred_element_type=jnp.float32)
        + b_ref[...][None, :])
    xt1_ref[...] = x1 * dinv[:, None]


def _t3_body(f_ref, g0_ref, g1_ref, dvg_ref, wb_ref, bb_ref, w1_ref, b1_ref,
             wf_ref, bf_ref, wo_ref, bo_ref, out_ref):
    base = jax.nn.relu(
        jnp.dot(f_ref[...], wb_ref[...], preferred_element_type=jnp.float32)
        + bb_ref[...][None, :])
    dvg = dvg_ref[...]
    aggg = (g0_ref[...] + g1_ref[...]) * dvg[:, None]
    graph = jax.nn.relu(
        jnp.dot(aggg, w1_ref[...], preferred_element_type=jnp.float32)
        + b1_ref[...][None, :])
    fusion = jax.nn.relu(
        jnp.dot(base, wf_ref[0:BASE_OUT, :], preferred_element_type=jnp.float32)
        + jnp.dot(graph, wf_ref[BASE_OUT:, :], preferred_element_type=jnp.float32)
        + bf_ref[...][None, :])
    out_ref[...] = jax.nn.sigmoid(
        jnp.dot(fusion, wo_ref[...], preferred_element_type=jnp.float32)
        + bo_ref[...][None, :])


def kernel(movieId, features, node_features, edge_index, Wb, bb, W_gcn0,
           b_gcn0, W_gcn1, b_gcn1, Wf, bf, Wo, bo):
    src = edge_index[0].astype(jnp.int32).reshape(NW, CHUNKS, K)
    dst = edge_index[1].astype(jnp.int32).reshape(NW, CHUNKS, K)

    dst_d = edge_index[1].astype(jnp.int32).reshape(NW, CHD, KD)
    deg_partials = _deg_kernel(dst_d).reshape(NC, NPAD)

    dinv, xt0 = pl.pallas_call(
        _t1_body,
        out_shape=[
            jax.ShapeDtypeStruct((N,), jnp.float32),
            jax.ShapeDtypeStruct((N, D), jnp.float32),
        ],
    )(deg_partials, node_features)

    p = _prop_kernel(xt0, src, dst)

    xt1 = pl.pallas_call(
        _t2_body,
        out_shape=jax.ShapeDtypeStruct((N, GCN_UNITS), jnp.float32),
    )(p, dinv, W_gcn0, b_gcn0)

    g, dvg = _prop2_kernel(xt1, src, dst, movieId, dinv)

    out = pl.pallas_call(
        _t3_body,
        out_shape=jax.ShapeDtypeStruct((B, 1), jnp.float32),
    )(features, g[0], g[1], dvg, Wb, bb, W_gcn1, b_gcn1, Wf, bf, Wo, bo)
    return out
